# Initial kernel scaffold; baseline (speedup 1.0000x reference)
import jax
import jax.numpy as jnp
from jax.experimental import pallas as pl


def kernel(iwords, owords, nwords, emb_i, emb_o):
    raise NotImplementedError("write your pallas kernel here")


# speedup vs baseline: 6.8362x; 6.8362x over previous
import functools

import jax
import jax.numpy as jnp
from jax import lax
from jax.experimental import pallas as pl
from jax.experimental.pallas import tpu as pltpu
from jax.experimental.pallas import tpu_sc as plsc

VOCAB = 100000
DIM = 128
BATCH = 4096
N_NEGS = 64

NC = 2
NS = 16
L = 16
NW = NC * NS
B_PER_W = BATCH // NW
CHUNK_B = 8
N_CHUNKS = B_PER_W // CHUNK_B
ROWS_PER_CHUNK = CHUNK_B * N_NEGS
DQ = DIM // L


def _lanesum(x):
    lane = lax.iota(jnp.int32, L)
    for sh in (8, 4, 2, 1):
        x = x + x.at[lane ^ sh].get(mode="promise_in_bounds")
    return x


def _dot_row(ref, row, iv_regs):
    acc = ref[row, pl.ds(0, L)] * iv_regs[0]
    for q in range(1, DQ):
        acc = acc + ref[row, pl.ds(q * L, L)] * iv_regs[q]
    return _lanesum(acc)


def _sc_body(iw_hbm, ow_hbm, nw_hbm, ei_hbm, eo_hbm, os_hbm, ns_hbm,
             idx_i, idx_o, idx_n, ivec, ovec, nvec, osc, nsc, sem):
    wid = lax.axis_index("s") * NC + lax.axis_index("c")
    base = wid * B_PER_W
    lane = lax.iota(jnp.int32, L)

    pltpu.sync_copy(iw_hbm.at[pl.ds(base, B_PER_W)], idx_i)
    pltpu.sync_copy(ow_hbm.at[pl.ds(base, B_PER_W)], idx_o)
    cp_i = pltpu.async_copy(ei_hbm.at[idx_i], ivec, sem)
    cp_o = pltpu.async_copy(eo_hbm.at[idx_o], ovec, sem)
    cp_i.wait()
    cp_o.wait()

    def ogroup(g2, _):
        svec = jnp.zeros((L,), jnp.float32)
        for u in range(L):
            b = g2 * L + u
            iv_regs = [ivec[b, pl.ds(q * L, L)] for q in range(DQ)]
            s = _dot_row(ovec, b, iv_regs)
            svec = jnp.where(lane == u, s, svec)
        osc[pl.ds(g2 * L, L)] = svec
        return _

    lax.fori_loop(0, B_PER_W // L, ogroup, None)
    pltpu.sync_copy(osc, os_hbm.at[pl.ds(base, B_PER_W)])

    def chunk(g, _):
        nrow0 = wid * (B_PER_W * N_NEGS // DIM) + g * (ROWS_PER_CHUNK // DIM)
        pltpu.sync_copy(nw_hbm.at[pl.ds(nrow0, ROWS_PER_CHUNK // DIM)], idx_n)
        cps = [
            pltpu.async_copy(
                eo_hbm.at[idx_n.at[k]], nvec.at[pl.ds(k * DIM, DIM)], sem)
            for k in range(ROWS_PER_CHUNK // DIM)
        ]
        for cp in cps:
            cp.wait()

        def bloop(b2, _):
            b = g * CHUNK_B + b2
            iv_regs = [ivec[b, pl.ds(q * L, L)] for q in range(DQ)]

            def jgroup(jj, _):
                svec = jnp.zeros((L,), jnp.float32)
                for u in range(L):
                    r = b2 * N_NEGS + jj * L + u
                    s = _dot_row(nvec, r, iv_regs)
                    svec = jnp.where(lane == u, s, svec)
                nsc[pl.ds(b2 * N_NEGS + jj * L, L)] = svec
                return _

            lax.fori_loop(0, N_NEGS // L, jgroup, None)
            return _

        lax.fori_loop(0, CHUNK_B, bloop, None)
        pltpu.sync_copy(
            nsc, ns_hbm.at[pl.ds((base + g * CHUNK_B) * N_NEGS, ROWS_PER_CHUNK)])
        return _

    lax.fori_loop(0, N_CHUNKS, chunk, None)


_sc_scores = functools.partial(
    pl.kernel,
    out_type=[
        jax.ShapeDtypeStruct((BATCH,), jnp.float32),
        jax.ShapeDtypeStruct((BATCH * N_NEGS,), jnp.float32),
    ],
    mesh=plsc.VectorSubcoreMesh(core_axis_name="c", subcore_axis_name="s"),
    scratch_types=[
        pltpu.VMEM((B_PER_W,), jnp.int32),
        pltpu.VMEM((B_PER_W,), jnp.int32),
        pltpu.VMEM((ROWS_PER_CHUNK // DIM, DIM), jnp.int32),
        pltpu.VMEM((B_PER_W, DIM), jnp.float32),
        pltpu.VMEM((B_PER_W, DIM), jnp.float32),
        pltpu.VMEM((ROWS_PER_CHUNK, DIM), jnp.float32),
        pltpu.VMEM((B_PER_W,), jnp.float32),
        pltpu.VMEM((ROWS_PER_CHUNK,), jnp.float32),
        pltpu.SemaphoreType.DMA,
    ],
)(_sc_body)


def _loss_body(os_ref, ns_ref, out_ref):
    o = os_ref[...]
    n = ns_ref[...]

    def logsig(x):
        return jnp.minimum(x, 0.0) - jnp.log(1.0 + jnp.exp(-jnp.abs(x)))

    tot = jnp.sum(logsig(o)) + jnp.sum(logsig(-n))
    out_ref[0, 0] = -tot / jnp.float32(BATCH)


_loss = pl.pallas_call(
    _loss_body,
    out_shape=jax.ShapeDtypeStruct((1, 1), jnp.float32),
    out_specs=pl.BlockSpec(memory_space=pltpu.SMEM),
)


def kernel(iwords, owords, nwords, emb_i, emb_o):
    nw2 = nwords.reshape(BATCH * N_NEGS // DIM, DIM)
    osc, nsc = _sc_scores(iwords, owords, nw2, emb_i, emb_o)
    loss = _loss(osc.reshape(BATCH // DIM, DIM),
                 nsc.reshape(BATCH * N_NEGS // DIM, DIM))
    return loss[0, 0]
